# Initial kernel scaffold; baseline (speedup 1.0000x reference)
import functools

import jax
import jax.numpy as jnp
from jax import lax
from jax.experimental import pallas as pl
from jax.experimental.pallas import tpu as pltpu
from jax.experimental.pallas import tpu_sc as plsc

_N = 10000
_D = 128
_E = 320000
_NP = 10240
_NC = 2
_NS = 16
_NW = _NC * _NS
_CB = 128
_CH = -(-_E // (_NW * _CB))
_EP = _NW * _CB * _CH
_RPS = _NP // _NS
_DW = 16
_R = 1024

_sc_mesh = plsc.VectorSubcoreMesh(core_axis_name="c", subcore_axis_name="s")


@functools.partial(
    pl.kernel,
    mesh=_sc_mesh,
    out_type=jax.ShapeDtypeStruct((_NC, _NP, _DW), jnp.float32),
    scratch_types=[
        pltpu.VMEM((_CH, _CB), jnp.int32),
        pltpu.VMEM((_CB, _DW), jnp.float32),
        pltpu.VMEM_SHARED((_NP, _DW), jnp.float32),
    ],
)
def _sc_degree(dst_hbm, ones_hbm, zero_hbm, out_hbm, dst_v, ones_v, acc_sh):
    c = lax.axis_index("c")
    s = lax.axis_index("s")
    wid = s * _NC + c
    pltpu.sync_copy(dst_hbm.at[wid], dst_v)
    pltpu.sync_copy(ones_hbm, ones_v)
    pltpu.sync_copy(zero_hbm, acc_sh.at[pl.ds(s * _RPS, _RPS)])
    plsc.subcore_barrier()

    def body(j, carry):
        pltpu.sync_copy(ones_v, acc_sh.at[dst_v.at[j]], add=True)
        return carry

    lax.fori_loop(0, _CH, body, 0)
    plsc.subcore_barrier()
    pltpu.sync_copy(acc_sh.at[pl.ds(s * _RPS, _RPS)],
                    out_hbm.at[c].at[pl.ds(s * _RPS, _RPS)])


@functools.partial(
    pl.kernel,
    mesh=_sc_mesh,
    out_type=jax.ShapeDtypeStruct((_NC, _NP, _D), jnp.float32),
    scratch_types=[
        pltpu.VMEM((_CH, _CB), jnp.int32),
        pltpu.VMEM((_CH, _CB), jnp.int32),
        pltpu.VMEM((_CB, _D), jnp.float32),
        pltpu.VMEM_SHARED((_NP, _D), jnp.float32),
    ],
)
def _sc_aggregate(y_hbm, src_hbm, dst_hbm, zero_hbm, out_hbm,
                  src_v, dst_v, rows_v, acc_sh):
    c = lax.axis_index("c")
    s = lax.axis_index("s")
    wid = s * _NC + c
    pltpu.sync_copy(src_hbm.at[wid], src_v)
    pltpu.sync_copy(dst_hbm.at[wid], dst_v)
    pltpu.sync_copy(zero_hbm, acc_sh.at[pl.ds(s * _RPS, _RPS)])
    plsc.subcore_barrier()

    def body(j, carry):
        pltpu.sync_copy(y_hbm.at[src_v.at[j]], rows_v)
        pltpu.sync_copy(rows_v, acc_sh.at[dst_v.at[j]], add=True)
        return carry

    lax.fori_loop(0, _CH, body, 0)
    plsc.subcore_barrier()
    pltpu.sync_copy(acc_sh.at[pl.ds(s * _RPS, _RPS)],
                    out_hbm.at[c].at[pl.ds(s * _RPS, _RPS)])


def _tc_prep(degp, x_p, W):
    def body(deg_ref, x_ref, w_ref, y_ref, xw_ref, dinv_ref):
        deg = deg_ref[0] + deg_ref[1] + 1.0
        dinv = lax.rsqrt(deg)
        xw = jnp.dot(x_ref[...], w_ref[...], preferred_element_type=jnp.float32)
        y_ref[...] = dinv[:, 0:1] * xw
        xw_ref[...] = xw
        dinv_ref[...] = dinv

    return pl.pallas_call(
        body,
        grid=(_NP // _R,),
        in_specs=[
            pl.BlockSpec((_NC, _R, _DW), lambda i: (0, i, 0)),
            pl.BlockSpec((_R, _D), lambda i: (i, 0)),
            pl.BlockSpec((_D, _D), lambda i: (0, 0)),
        ],
        out_specs=[
            pl.BlockSpec((_R, _D), lambda i: (i, 0)),
            pl.BlockSpec((_R, _D), lambda i: (i, 0)),
            pl.BlockSpec((_R, _DW), lambda i: (i, 0)),
        ],
        out_shape=[
            jax.ShapeDtypeStruct((_NP, _D), jnp.float32),
            jax.ShapeDtypeStruct((_NP, _D), jnp.float32),
            jax.ShapeDtypeStruct((_NP, _DW), jnp.float32),
        ],
    )(degp, x_p, W)


def _tc_mid(parts, xw1, dinv, b1, W2):
    def body(p_ref, xw_ref, dinv_ref, b_ref, w_ref, y_ref, xw2_ref):
        dv = dinv_ref[:, 0:1]
        h = dv * (p_ref[0] + p_ref[1]) + (dv * dv) * xw_ref[...] + b_ref[...]
        h = jnp.maximum(h, 0.0)
        xw2 = jnp.dot(h, w_ref[...], preferred_element_type=jnp.float32)
        y_ref[...] = dv * xw2
        xw2_ref[...] = xw2

    return pl.pallas_call(
        body,
        grid=(_NP // _R,),
        in_specs=[
            pl.BlockSpec((_NC, _R, _D), lambda i: (0, i, 0)),
            pl.BlockSpec((_R, _D), lambda i: (i, 0)),
            pl.BlockSpec((_R, _DW), lambda i: (i, 0)),
            pl.BlockSpec((1, _D), lambda i: (0, 0)),
            pl.BlockSpec((_D, _D), lambda i: (0, 0)),
        ],
        out_specs=[
            pl.BlockSpec((_R, _D), lambda i: (i, 0)),
            pl.BlockSpec((_R, _D), lambda i: (i, 0)),
        ],
        out_shape=[
            jax.ShapeDtypeStruct((_NP, _D), jnp.float32),
            jax.ShapeDtypeStruct((_NP, _D), jnp.float32),
        ],
    )(parts, xw1, dinv, b1, W2)


def _tc_final(parts, xw2, dinv, b2):
    def body(p_ref, xw_ref, dinv_ref, b_ref, o_ref):
        dv = dinv_ref[:, 0:1]
        o_ref[...] = dv * (p_ref[0] + p_ref[1]) + (dv * dv) * xw_ref[...] + b_ref[...]

    return pl.pallas_call(
        body,
        grid=(_NP // _R,),
        in_specs=[
            pl.BlockSpec((_NC, _R, _D), lambda i: (0, i, 0)),
            pl.BlockSpec((_R, _D), lambda i: (i, 0)),
            pl.BlockSpec((_R, _DW), lambda i: (i, 0)),
            pl.BlockSpec((1, _D), lambda i: (0, 0)),
        ],
        out_specs=pl.BlockSpec((_R, _D), lambda i: (i, 0)),
        out_shape=jax.ShapeDtypeStruct((_NP, _D), jnp.float32),
    )(parts, xw2, dinv, b2)


def kernel(x, edge_index, W1, b1, W2, b2):
    src = edge_index[0].astype(jnp.int32)
    dst = edge_index[1].astype(jnp.int32)
    pad = jnp.full((_EP - _E,), _N, dtype=jnp.int32)
    src_p = jnp.concatenate([src, pad]).reshape(_NW, _CH, _CB)
    dst_p = jnp.concatenate([dst, pad]).reshape(_NW, _CH, _CB)
    x_p = jnp.pad(x, ((0, _NP - _N), (0, 0)))
    ones_dw = jnp.ones((_CB, _DW), jnp.float32)
    zero_dw = jnp.zeros((_RPS, _DW), jnp.float32)
    zero_d = jnp.zeros((_RPS, _D), jnp.float32)

    degp = _sc_degree(dst_p, ones_dw, zero_dw)
    y1, xw1, dinv = _tc_prep(degp, x_p, W1)
    p1 = _sc_aggregate(y1, src_p, dst_p, zero_d)
    y2, xw2 = _tc_mid(p1, xw1, dinv, b1.reshape(1, _D), W2)
    p2 = _sc_aggregate(y2, src_p, dst_p, zero_d)
    out = _tc_final(p2, xw2, dinv, b2.reshape(1, _D))
    return out[:_N]


# speedup vs baseline: 12.3792x; 12.3792x over previous
import functools

import jax
import jax.numpy as jnp
from jax import lax
from jax.experimental import pallas as pl
from jax.experimental.pallas import tpu as pltpu
from jax.experimental.pallas import tpu_sc as plsc

_N = 10000
_D = 128
_E = 320000
_NP = 10240
_NC = 2
_NS = 16
_NW = _NC * _NS
_CB = 128
_CH = -(-_E // (_NW * _CB))
_EP = _NW * _CB * _CH
_RPS = _NP // _NS
_DW = 128
_R = 1024

@functools.cache
def _sc_kernels():
    mesh = plsc.VectorSubcoreMesh(core_axis_name="c", subcore_axis_name="s")

    @functools.partial(
        pl.kernel,
        mesh=mesh,
        out_type=jax.ShapeDtypeStruct((_NC, _NP, _DW), jnp.float32),
        scratch_types=[
            pltpu.VMEM((_CH, _CB), jnp.int32),
            pltpu.VMEM((_CB, _DW), jnp.float32),
            pltpu.VMEM_SHARED((_NP, _DW), jnp.float32),
        ],
    )
    def sc_degree(dst_hbm, ones_hbm, zero_hbm, out_hbm, dst_v, ones_v, acc_sh):
        c = lax.axis_index("c")
        s = lax.axis_index("s")
        wid = s * _NC + c
        pltpu.sync_copy(dst_hbm.at[wid], dst_v)
        pltpu.sync_copy(ones_hbm, ones_v)
        pltpu.sync_copy(zero_hbm, acc_sh.at[pl.ds(s * _RPS, _RPS)])
        plsc.subcore_barrier()

        def body(j, carry):
            pltpu.sync_copy(ones_v, acc_sh.at[dst_v.at[j]], add=True)
            return carry

        lax.fori_loop(0, _CH, body, 0)
        plsc.subcore_barrier()
        pltpu.sync_copy(acc_sh.at[pl.ds(s * _RPS, _RPS)],
                        out_hbm.at[c].at[pl.ds(s * _RPS, _RPS)])

    @functools.partial(
        pl.kernel,
        mesh=mesh,
        out_type=jax.ShapeDtypeStruct((_NC, _NP, _D), jnp.float32),
        scratch_types=[
            pltpu.VMEM((_CH, _CB), jnp.int32),
            pltpu.VMEM((_CH, _CB), jnp.int32),
            pltpu.VMEM((_CB, _D), jnp.float32),
            pltpu.VMEM_SHARED((_NP, _D), jnp.float32),
        ],
    )
    def sc_aggregate(y_hbm, src_hbm, dst_hbm, zero_hbm, out_hbm,
                     src_v, dst_v, rows_v, acc_sh):
        c = lax.axis_index("c")
        s = lax.axis_index("s")
        wid = s * _NC + c
        pltpu.sync_copy(src_hbm.at[wid], src_v)
        pltpu.sync_copy(dst_hbm.at[wid], dst_v)
        pltpu.sync_copy(zero_hbm, acc_sh.at[pl.ds(s * _RPS, _RPS)])
        plsc.subcore_barrier()

        def body(j, carry):
            pltpu.sync_copy(y_hbm.at[src_v.at[j]], rows_v)
            pltpu.sync_copy(rows_v, acc_sh.at[dst_v.at[j]], add=True)
            return carry

        lax.fori_loop(0, _CH, body, 0)
        plsc.subcore_barrier()
        pltpu.sync_copy(acc_sh.at[pl.ds(s * _RPS, _RPS)],
                        out_hbm.at[c].at[pl.ds(s * _RPS, _RPS)])

    return sc_degree, sc_aggregate


def _tc_prep(degp, x_p, W):
    def body(deg_ref, x_ref, w_ref, y_ref, xw_ref, dinv_ref):
        deg = deg_ref[0] + deg_ref[1] + 1.0
        dinv = lax.rsqrt(deg)
        xw = jnp.dot(x_ref[...], w_ref[...], preferred_element_type=jnp.float32)
        y_ref[...] = dinv * xw
        xw_ref[...] = xw
        dinv_ref[...] = dinv

    return pl.pallas_call(
        body,
        grid=(_NP // _R,),
        in_specs=[
            pl.BlockSpec((_NC, _R, _DW), lambda i: (0, i, 0)),
            pl.BlockSpec((_R, _D), lambda i: (i, 0)),
            pl.BlockSpec((_D, _D), lambda i: (0, 0)),
        ],
        out_specs=[
            pl.BlockSpec((_R, _D), lambda i: (i, 0)),
            pl.BlockSpec((_R, _D), lambda i: (i, 0)),
            pl.BlockSpec((_R, _DW), lambda i: (i, 0)),
        ],
        out_shape=[
            jax.ShapeDtypeStruct((_NP, _D), jnp.float32),
            jax.ShapeDtypeStruct((_NP, _D), jnp.float32),
            jax.ShapeDtypeStruct((_NP, _DW), jnp.float32),
        ],
    )(degp, x_p, W)


def _tc_mid(parts, xw1, dinv, b1, W2):
    def body(p_ref, xw_ref, dinv_ref, b_ref, w_ref, y_ref, xw2_ref):
        dv = dinv_ref[...]
        h = dv * (p_ref[0] + p_ref[1]) + (dv * dv) * xw_ref[...] + b_ref[...]
        h = jnp.maximum(h, 0.0)
        xw2 = jnp.dot(h, w_ref[...], preferred_element_type=jnp.float32)
        y_ref[...] = dv * xw2
        xw2_ref[...] = xw2

    return pl.pallas_call(
        body,
        grid=(_NP // _R,),
        in_specs=[
            pl.BlockSpec((_NC, _R, _D), lambda i: (0, i, 0)),
            pl.BlockSpec((_R, _D), lambda i: (i, 0)),
            pl.BlockSpec((_R, _DW), lambda i: (i, 0)),
            pl.BlockSpec((1, _D), lambda i: (0, 0)),
            pl.BlockSpec((_D, _D), lambda i: (0, 0)),
        ],
        out_specs=[
            pl.BlockSpec((_R, _D), lambda i: (i, 0)),
            pl.BlockSpec((_R, _D), lambda i: (i, 0)),
        ],
        out_shape=[
            jax.ShapeDtypeStruct((_NP, _D), jnp.float32),
            jax.ShapeDtypeStruct((_NP, _D), jnp.float32),
        ],
    )(parts, xw1, dinv, b1, W2)


def _tc_final(parts, xw2, dinv, b2):
    def body(p_ref, xw_ref, dinv_ref, b_ref, o_ref):
        dv = dinv_ref[...]
        o_ref[...] = dv * (p_ref[0] + p_ref[1]) + (dv * dv) * xw_ref[...] + b_ref[...]

    return pl.pallas_call(
        body,
        grid=(_NP // _R,),
        in_specs=[
            pl.BlockSpec((_NC, _R, _D), lambda i: (0, i, 0)),
            pl.BlockSpec((_R, _D), lambda i: (i, 0)),
            pl.BlockSpec((_R, _DW), lambda i: (i, 0)),
            pl.BlockSpec((1, _D), lambda i: (0, 0)),
        ],
        out_specs=pl.BlockSpec((_R, _D), lambda i: (i, 0)),
        out_shape=jax.ShapeDtypeStruct((_NP, _D), jnp.float32),
    )(parts, xw2, dinv, b2)


def kernel(x, edge_index, W1, b1, W2, b2):
    src = edge_index[0].astype(jnp.int32)
    dst = edge_index[1].astype(jnp.int32)
    pad = jnp.full((_EP - _E,), _N, dtype=jnp.int32)
    src_p = jnp.concatenate([src, pad]).reshape(_NW, _CH, _CB)
    dst_p = jnp.concatenate([dst, pad]).reshape(_NW, _CH, _CB)
    x_p = jnp.pad(x, ((0, _NP - _N), (0, 0)))
    ones_dw = jnp.ones((_CB, _DW), jnp.float32)
    zero_d = jnp.zeros((_RPS, _D), jnp.float32)

    sc_degree, sc_aggregate = _sc_kernels()
    degp = sc_degree(dst_p, ones_dw, zero_d)
    y1, xw1, dinv = _tc_prep(degp, x_p, W1)
    p1 = sc_aggregate(y1, src_p, dst_p, zero_d)
    y2, xw2 = _tc_mid(p1, xw1, dinv, b1.reshape(1, _D), W2)
    p2 = sc_aggregate(y2, src_p, dst_p, zero_d)
    out = _tc_final(p2, xw2, dinv, b2.reshape(1, _D))
    return out[:_N]


# speedup vs baseline: 13.4082x; 1.0831x over previous
import functools

import jax
import jax.numpy as jnp
from jax import lax
from jax.experimental import pallas as pl
from jax.experimental.pallas import tpu as pltpu
from jax.experimental.pallas import tpu_sc as plsc

_N = 10000
_D = 128
_E = 320000
_NP = 10240
_NC = 2
_NS = 16
_NW = _NC * _NS
_CB = 128
_CH = -(-_E // (_NW * _CB))
_CHH = (_CH + 1) // 2
_EP = _NW * _CB * _CH
_RPS = _NP // _NS
_DW = 128
_R = 1024

@functools.cache
def _sc_kernels():
    mesh = plsc.VectorSubcoreMesh(core_axis_name="c", subcore_axis_name="s")

    @functools.partial(
        pl.kernel,
        mesh=mesh,
        out_type=jax.ShapeDtypeStruct((_NC, _NP, _DW), jnp.float32),
        scratch_types=[
            pltpu.VMEM((_CH, _CB), jnp.int32),
            pltpu.VMEM((_CB, _DW), jnp.float32),
            pltpu.VMEM_SHARED((_NP, _DW), jnp.float32),
        ],
    )
    def sc_degree(dst_hbm, ones_hbm, zero_hbm, out_hbm, dst_v, ones_v, acc_sh):
        c = lax.axis_index("c")
        s = lax.axis_index("s")
        wid = s * _NC + c
        pltpu.sync_copy(dst_hbm.at[wid], dst_v)
        pltpu.sync_copy(ones_hbm, ones_v)
        pltpu.sync_copy(zero_hbm, acc_sh.at[pl.ds(s * _RPS, _RPS)])
        plsc.subcore_barrier()

        def body(j, carry):
            pltpu.sync_copy(ones_v, acc_sh.at[dst_v.at[j]], add=True)
            return carry

        lax.fori_loop(0, _CH, body, 0)
        plsc.subcore_barrier()
        pltpu.sync_copy(acc_sh.at[pl.ds(s * _RPS, _RPS)],
                        out_hbm.at[c].at[pl.ds(s * _RPS, _RPS)])

    @functools.partial(
        pl.kernel,
        mesh=mesh,
        out_type=jax.ShapeDtypeStruct((_NC, _NP, _D), jnp.float32),
        scratch_types=[
            pltpu.VMEM((_CHH, _CB), jnp.int32),
            pltpu.VMEM((_CHH, _CB), jnp.int32),
            pltpu.VMEM((_CB, _D), jnp.float32),
            pltpu.VMEM((_CB, _D), jnp.float32),
            pltpu.VMEM_SHARED((_NP, _D), jnp.float32),
            pltpu.SemaphoreType.DMA,
            pltpu.SemaphoreType.DMA,
            pltpu.SemaphoreType.DMA,
            pltpu.SemaphoreType.DMA,
        ],
    )
    def sc_aggregate(y_hbm, src_hbm, dst_hbm, zero_hbm, out_hbm,
                     src_v, dst_v, rows0, rows1, acc_sh, gs0, gs1, ss0, ss1):
        c = lax.axis_index("c")
        s = lax.axis_index("s")
        wid = s * _NC + c
        pltpu.sync_copy(zero_hbm, acc_sh.at[pl.ds(s * _RPS, _RPS)])
        plsc.subcore_barrier()

        for base, nch in ((0, _CHH), (_CHH, _CH - _CHH)):
            pltpu.sync_copy(src_hbm.at[wid].at[pl.ds(base, nch)],
                            src_v.at[pl.ds(0, nch)])
            pltpu.sync_copy(dst_hbm.at[wid].at[pl.ds(base, nch)],
                            dst_v.at[pl.ds(0, nch)])
            last = nch - 1
            pltpu.async_copy(y_hbm.at[src_v.at[0]], rows0, gs0)
            pltpu.async_copy(y_hbm.at[src_v.at[1]], rows1, gs1)

            def body(j, carry):
                e0 = 2 * j
                e1 = 2 * j + 1
                pltpu.make_async_copy(y_hbm.at[src_v.at[e0]], rows0, gs0).wait()
                pltpu.async_copy(rows0, acc_sh.at[dst_v.at[e0]], ss0, add=True)
                pltpu.make_async_copy(y_hbm.at[src_v.at[e1]], rows1, gs1).wait()
                pltpu.async_copy(rows1, acc_sh.at[dst_v.at[e1]], ss1, add=True)
                pltpu.make_async_copy(rows0, acc_sh.at[dst_v.at[e0]], ss0).wait()
                pltpu.async_copy(
                    y_hbm.at[src_v.at[jnp.minimum(e0 + 2, last)]], rows0, gs0)
                pltpu.make_async_copy(rows1, acc_sh.at[dst_v.at[e1]], ss1).wait()
                pltpu.async_copy(
                    y_hbm.at[src_v.at[jnp.minimum(e1 + 2, last)]], rows1, gs1)
                return carry

            lax.fori_loop(0, nch // 2, body, 0)
            if nch % 2 == 1:
                pltpu.make_async_copy(y_hbm.at[src_v.at[last]], rows0, gs0).wait()
                pltpu.sync_copy(rows0, acc_sh.at[dst_v.at[last]], add=True)
                pltpu.make_async_copy(y_hbm.at[src_v.at[last]], rows1, gs1).wait()
            else:
                pltpu.make_async_copy(y_hbm.at[src_v.at[last]], rows0, gs0).wait()
                pltpu.make_async_copy(y_hbm.at[src_v.at[last]], rows1, gs1).wait()
        plsc.subcore_barrier()
        pltpu.sync_copy(acc_sh.at[pl.ds(s * _RPS, _RPS)],
                        out_hbm.at[c].at[pl.ds(s * _RPS, _RPS)])

    return sc_degree, sc_aggregate


def _tc_prep(degp, x_p, W):
    def body(deg_ref, x_ref, w_ref, y_ref, xw_ref, dinv_ref):
        deg = deg_ref[0] + deg_ref[1] + 1.0
        dinv = lax.rsqrt(deg)
        xw = jnp.dot(x_ref[...], w_ref[...], preferred_element_type=jnp.float32)
        y_ref[...] = dinv * xw
        xw_ref[...] = xw
        dinv_ref[...] = dinv

    return pl.pallas_call(
        body,
        grid=(_NP // _R,),
        in_specs=[
            pl.BlockSpec((_NC, _R, _DW), lambda i: (0, i, 0)),
            pl.BlockSpec((_R, _D), lambda i: (i, 0)),
            pl.BlockSpec((_D, _D), lambda i: (0, 0)),
        ],
        out_specs=[
            pl.BlockSpec((_R, _D), lambda i: (i, 0)),
            pl.BlockSpec((_R, _D), lambda i: (i, 0)),
            pl.BlockSpec((_R, _DW), lambda i: (i, 0)),
        ],
        out_shape=[
            jax.ShapeDtypeStruct((_NP, _D), jnp.float32),
            jax.ShapeDtypeStruct((_NP, _D), jnp.float32),
            jax.ShapeDtypeStruct((_NP, _DW), jnp.float32),
        ],
    )(degp, x_p, W)


def _tc_mid(parts, xw1, dinv, b1, W2):
    def body(p_ref, xw_ref, dinv_ref, b_ref, w_ref, y_ref, xw2_ref):
        dv = dinv_ref[...]
        h = dv * (p_ref[0] + p_ref[1]) + (dv * dv) * xw_ref[...] + b_ref[...]
        h = jnp.maximum(h, 0.0)
        xw2 = jnp.dot(h, w_ref[...], preferred_element_type=jnp.float32)
        y_ref[...] = dv * xw2
        xw2_ref[...] = xw2

    return pl.pallas_call(
        body,
        grid=(_NP // _R,),
        in_specs=[
            pl.BlockSpec((_NC, _R, _D), lambda i: (0, i, 0)),
            pl.BlockSpec((_R, _D), lambda i: (i, 0)),
            pl.BlockSpec((_R, _DW), lambda i: (i, 0)),
            pl.BlockSpec((1, _D), lambda i: (0, 0)),
            pl.BlockSpec((_D, _D), lambda i: (0, 0)),
        ],
        out_specs=[
            pl.BlockSpec((_R, _D), lambda i: (i, 0)),
            pl.BlockSpec((_R, _D), lambda i: (i, 0)),
        ],
        out_shape=[
            jax.ShapeDtypeStruct((_NP, _D), jnp.float32),
            jax.ShapeDtypeStruct((_NP, _D), jnp.float32),
        ],
    )(parts, xw1, dinv, b1, W2)


def _tc_final(parts, xw2, dinv, b2):
    def body(p_ref, xw_ref, dinv_ref, b_ref, o_ref):
        dv = dinv_ref[...]
        o_ref[...] = dv * (p_ref[0] + p_ref[1]) + (dv * dv) * xw_ref[...] + b_ref[...]

    return pl.pallas_call(
        body,
        grid=(_NP // _R,),
        in_specs=[
            pl.BlockSpec((_NC, _R, _D), lambda i: (0, i, 0)),
            pl.BlockSpec((_R, _D), lambda i: (i, 0)),
            pl.BlockSpec((_R, _DW), lambda i: (i, 0)),
            pl.BlockSpec((1, _D), lambda i: (0, 0)),
        ],
        out_specs=pl.BlockSpec((_R, _D), lambda i: (i, 0)),
        out_shape=jax.ShapeDtypeStruct((_NP, _D), jnp.float32),
    )(parts, xw2, dinv, b2)


def kernel(x, edge_index, W1, b1, W2, b2):
    src = edge_index[0].astype(jnp.int32)
    dst = edge_index[1].astype(jnp.int32)
    pad = jnp.full((_EP - _E,), _N, dtype=jnp.int32)
    src_p = jnp.concatenate([src, pad]).reshape(_NW, _CH, _CB)
    dst_p = jnp.concatenate([dst, pad]).reshape(_NW, _CH, _CB)
    x_p = jnp.pad(x, ((0, _NP - _N), (0, 0)))
    ones_dw = jnp.ones((_CB, _DW), jnp.float32)
    zero_d = jnp.zeros((_RPS, _D), jnp.float32)

    sc_degree, sc_aggregate = _sc_kernels()
    degp = sc_degree(dst_p, ones_dw, zero_d)
    y1, xw1, dinv = _tc_prep(degp, x_p, W1)
    p1 = sc_aggregate(y1, src_p, dst_p, zero_d)
    y2, xw2 = _tc_mid(p1, xw1, dinv, b1.reshape(1, _D), W2)
    p2 = sc_aggregate(y2, src_p, dst_p, zero_d)
    out = _tc_final(p2, xw2, dinv, b2.reshape(1, _D))
    return out[:_N]
